# baseline (speedup 1.0000x reference)
import functools

import jax
import jax.numpy as jnp
from jax import lax
from jax.experimental import pallas as pl
from jax.experimental.pallas import tpu as pltpu
from jax.experimental.pallas import tpu_sc as plsc

N = 10000
E = 160000
IN = 256
HEADS = 8
OUT = 64
D = HEADS * OUT
HID = 64

ROWS = 400
NBLK = N // ROWS

NC = 2
NS = 16
NW = NC * NS
K = 64
EW = 5120
EPAD = NW * EW
NPAD = 10240
RPS = NPAD // NS
NCHUNK = EW // K

_MESH = plsc.VectorSubcoreMesh(core_axis_name="c", subcore_axis_name="s")
_SC_PARAMS = pltpu.CompilerParams(use_tc_tiling_on_sc=False,
                                  needs_layout_passes=False)


def _feat_elr_kernel(h_ref, w_ref, b_ref, feat_ref, elr_ref):
    f = jnp.dot(h_ref[...], w_ref[...], preferred_element_type=jnp.float32)
    feat_ref[...] = f
    elr_ref[...] = jnp.dot(f, b_ref[...], preferred_element_type=jnp.float32)


def _z_sem_kernel(n0_ref, n1_ref, d0_ref, d1_ref, m_ref, b0_ref, b1_ref,
                  sw1_ref, sb1_ref, sw2_ref, z0_ref, z1_ref, w_ref):
    num0 = n0_ref[0] + n0_ref[1]
    num1 = n1_ref[0] + n1_ref[1]
    den0 = d0_ref[0] + d0_ref[1] + 1e-9
    den1 = d1_ref[0] + d1_ref[1] + 1e-9
    dex0 = jnp.dot(1.0 / den0, m_ref[...], preferred_element_type=jnp.float32)
    dex1 = jnp.dot(1.0 / den1, m_ref[...], preferred_element_type=jnp.float32)
    z0 = num0 * dex0 + b0_ref[...]
    z1 = num1 * dex1 + b1_ref[...]
    z0 = jnp.where(z0 > 0, z0, jnp.exp(jnp.minimum(z0, 0.0)) - 1.0)
    z1 = jnp.where(z1 > 0, z1, jnp.exp(jnp.minimum(z1, 0.0)) - 1.0)
    z0_ref[...] = z0
    z1_ref[...] = z1
    t0 = jnp.dot(jnp.tanh(jnp.dot(z0, sw1_ref[...],
                                  preferred_element_type=jnp.float32)
                          + sb1_ref[...]),
                 sw2_ref[...], preferred_element_type=jnp.float32)
    t1 = jnp.dot(jnp.tanh(jnp.dot(z1, sw1_ref[...],
                                  preferred_element_type=jnp.float32)
                          + sb1_ref[...]),
                 sw2_ref[...], preferred_element_type=jnp.float32)
    w_ref[...] = jnp.concatenate(
        [jnp.sum(t0).reshape(1, 1, 1), jnp.sum(t1).reshape(1, 1, 1)], axis=2)


def _combine_kernel(z0_ref, z1_ref, w_ref, out_ref):
    w0 = w_ref[0, 0]
    w1 = w_ref[0, 1]
    m = jnp.maximum(w0, w1)
    e0 = jnp.exp(w0 - m)
    e1 = jnp.exp(w1 - m)
    beta0 = e0 / (e0 + e1)
    beta1 = e1 / (e0 + e1)
    out_ref[...] = beta0 * z0_ref[...] + beta1 * z1_ref[...]


def _edge_logits_body(ela_hbm, erb_hbm, src_hbm, dst_hbm, z16_hbm,
                      ee_hbm, den_hbm,
                      sidx, didx, ga, gb, eev, acc):
    cid = lax.axis_index("c")
    sid = lax.axis_index("s")
    wid = sid * NC + cid

    pltpu.sync_copy(z16_hbm, acc.at[pl.ds(sid * RPS, RPS)])
    plsc.subcore_barrier()

    @pl.loop(0, NCHUNK)
    def _(i):
        base = wid * EW + i * K
        pltpu.sync_copy(src_hbm.at[pl.ds(base, K)], sidx.at[0])
        pltpu.sync_copy(dst_hbm.at[pl.ds(base, K)], didx.at[0])
        pltpu.sync_copy(ela_hbm.at[sidx.at[0]], ga)
        pltpu.sync_copy(erb_hbm.at[didx.at[0]], gb)

        @pl.loop(0, K)
        def _(r):
            x = ga[r] + gb[r]
            x = jnp.maximum(x, 0.2 * x)
            eev[r] = jnp.exp(x)

        pltpu.sync_copy(eev, ee_hbm.at[pl.ds(base, K)])
        pltpu.sync_copy(eev, acc.at[didx.at[0]], add=True)

    plsc.subcore_barrier()
    pltpu.sync_copy(acc.at[pl.ds(sid * RPS, RPS)],
                    den_hbm.at[cid, pl.ds(sid * RPS, RPS)])


def _aggregate_body(f0_hbm, f1_hbm, f2_hbm, f3_hbm, src3_hbm, dst3_hbm,
                    ee_hbm, z128_hbm, num_hbm,
                    sidx, didx, g0, g1, g2, g3, ee0, ee1, ee2, ee3,
                    sem0, sem1, sem2, sem3, acc):
    cid = lax.axis_index("c")
    sid = lax.axis_index("s")
    wid = sid * NC + cid

    pltpu.sync_copy(src3_hbm.at[wid], sidx)
    pltpu.sync_copy(dst3_hbm.at[wid], didx)

    feats = (f0_hbm, f1_hbm, f2_hbm, f3_hbm)
    bufs = ((g0, ee0, sem0), (g1, ee1, sem1), (g2, ee2, sem2),
            (g3, ee3, sem3))
    NBUF = 4

    for p in range(4):
        fp = feats[p]

        pltpu.sync_copy(z128_hbm, acc.at[pl.ds(sid * RPS, RPS)])
        plsc.subcore_barrier()

        def issue(i, b):
            gb, eb, sb = bufs[b]
            pltpu.async_copy(fp.at[sidx.at[i]], gb, sb)
            pltpu.async_copy(ee_hbm.at[pl.ds(wid * EW + i * K, K)], eb, sb)

        for b0 in range(NBUF):
            issue(b0, b0)

        @pl.loop(0, NCHUNK, step=NBUF)
        def _(i2):
            for b in range(NBUF):
                i = i2 + b
                gb, eb, sb = bufs[b]
                pltpu.make_async_copy(fp.at[sidx.at[i]], gb, sb).wait()
                pltpu.make_async_copy(
                    ee_hbm.at[pl.ds(wid * EW + i * K, K)], eb, sb).wait()

                @pl.loop(0, K)
                def _(j):
                    jv = jnp.full((16,), j, jnp.int32)
                    m0 = plsc.load_gather(
                        eb, [jv, jnp.full((16,), 2 * p, jnp.int32)])
                    m1 = plsc.load_gather(
                        eb, [jv, jnp.full((16,), 2 * p + 1, jnp.int32)])
                    for v in range(8):
                        m = m0 if v < 4 else m1
                        c = v * 16
                        gb[j, pl.ds(c, 16)] = gb[j, pl.ds(c, 16)] * m

                pltpu.sync_copy(gb, acc.at[didx.at[i]], add=True)

                @pl.when(i + NBUF < NCHUNK)
                def _():
                    issue(i + NBUF, b)

        plsc.subcore_barrier()
        pltpu.sync_copy(acc.at[pl.ds(sid * RPS, RPS)],
                        num_hbm.at[p, cid, pl.ds(sid * RPS, RPS)])
        plsc.subcore_barrier()


def _edge_logits(ela, erb, src, dst, z16):
    kern = pl.kernel(
        _edge_logits_body,
        out_type=(jax.ShapeDtypeStruct((EPAD, 16), jnp.float32),
                  jax.ShapeDtypeStruct((NC, NPAD, 16), jnp.float32)),
        mesh=_MESH,
        scratch_types=[
            pltpu.VMEM((1, K), jnp.int32),
            pltpu.VMEM((1, K), jnp.int32),
            pltpu.VMEM((K, 16), jnp.float32),
            pltpu.VMEM((K, 16), jnp.float32),
            pltpu.VMEM((K, 16), jnp.float32),
            pltpu.VMEM_SHARED((NPAD, 16), jnp.float32),
        ],
        compiler_params=_SC_PARAMS,
    )
    return kern(ela, erb, src, dst, z16)


def _aggregate(f0, f1, f2, f3, src3, dst3, ee, z128):
    kern = pl.kernel(
        _aggregate_body,
        out_type=jax.ShapeDtypeStruct((4, NC, NPAD, 128), jnp.float32),
        mesh=_MESH,
        scratch_types=[
            pltpu.VMEM((NCHUNK, K), jnp.int32),
            pltpu.VMEM((NCHUNK, K), jnp.int32),
            pltpu.VMEM((K, 128), jnp.float32),
            pltpu.VMEM((K, 128), jnp.float32),
            pltpu.VMEM((K, 128), jnp.float32),
            pltpu.VMEM((K, 128), jnp.float32),
            pltpu.VMEM((K, 16), jnp.float32),
            pltpu.VMEM((K, 16), jnp.float32),
            pltpu.VMEM((K, 16), jnp.float32),
            pltpu.VMEM((K, 16), jnp.float32),
            pltpu.SemaphoreType.DMA,
            pltpu.SemaphoreType.DMA,
            pltpu.SemaphoreType.DMA,
            pltpu.SemaphoreType.DMA,
            pltpu.VMEM_SHARED((NPAD, 128), jnp.float32),
        ],
        compiler_params=_SC_PARAMS,
    )
    return kern(f0, f1, f2, f3, src3, dst3, ee, z128)


def _block_diag(a):
    return (jnp.eye(HEADS, dtype=a.dtype)[:, None, :] * a[:, :, None]).reshape(D, HEADS)


def kernel(H, edge_index0, edge_index1, W0, al0, ar0, b0, W1, al1, ar1, b1, sW1, sb1, sW2):
    h = H[0]
    wcat = jnp.concatenate([W0, W1], axis=1)

    bmat = jnp.zeros((2 * D, 64), dtype=jnp.float32)
    bmat = bmat.at[:D, 0:HEADS].set(_block_diag(al0))
    bmat = bmat.at[:D, 16:16 + HEADS].set(_block_diag(ar0))
    bmat = bmat.at[D:, 32:32 + HEADS].set(_block_diag(al1))
    bmat = bmat.at[D:, 48:48 + HEADS].set(_block_diag(ar1))

    feat01, elr = pl.pallas_call(
        _feat_elr_kernel,
        grid=(NBLK,),
        in_specs=[
            pl.BlockSpec((ROWS, IN), lambda i: (i, 0)),
            pl.BlockSpec((IN, 2 * D), lambda i: (0, 0)),
            pl.BlockSpec((2 * D, 64), lambda i: (0, 0)),
        ],
        out_specs=[
            pl.BlockSpec((ROWS, 2 * D), lambda i: (i, 0)),
            pl.BlockSpec((ROWS, 64), lambda i: (i, 0)),
        ],
        out_shape=[
            jax.ShapeDtypeStruct((N, 2 * D), jnp.float32),
            jax.ShapeDtypeStruct((N, 64), jnp.float32),
        ],
    )(h, wcat, bmat)

    ela0 = elr[:, 0:16]
    erb0 = jnp.pad(elr[:, 16:32], ((0, NPAD - N), (0, 0)))
    ela1 = elr[:, 32:48]
    erb1 = jnp.pad(elr[:, 48:64], ((0, NPAD - N), (0, 0)))

    pad = EPAD - E
    i32 = jnp.int32
    sink = N + jnp.arange(pad, dtype=i32) % (NPAD - N)
    src0 = jnp.concatenate([edge_index0[0], jnp.zeros((pad,), i32)])
    dst0 = jnp.concatenate([edge_index0[1], sink])
    src1 = jnp.concatenate([edge_index1[0], jnp.zeros((pad,), i32)])
    dst1 = jnp.concatenate([edge_index1[1], sink])

    z16 = jnp.zeros((RPS, 16), jnp.float32)
    z128 = jnp.zeros((RPS, 128), jnp.float32)

    ee0, den0 = _edge_logits(ela0, erb0, src0, dst0, z16)
    ee1, den1 = _edge_logits(ela1, erb1, src1, dst1, z16)

    nums = []
    for m, (srcm, dstm, eem) in enumerate(((src0, dst0, ee0),
                                           (src1, dst1, ee1))):
        fs = [lax.slice(feat01, (0, m * D + p * 128),
                        (N, m * D + (p + 1) * 128)) for p in range(4)]
        num4 = _aggregate(fs[0], fs[1], fs[2], fs[3],
                          srcm.reshape(NW, NCHUNK, K),
                          dstm.reshape(NW, NCHUNK, K), eem, z128)
        nums.append(jnp.concatenate(
            [num4[p] for p in range(4)], axis=2)[:, :N, :])

    den0 = den0[:, :N, :]
    den1 = den1[:, :N, :]

    mexp = jnp.zeros((16, D), jnp.float32)
    for hh in range(HEADS):
        mexp = mexp.at[hh, hh * OUT:(hh + 1) * OUT].set(1.0)

    z0, z1, wparts = pl.pallas_call(
        _z_sem_kernel,
        grid=(NBLK,),
        in_specs=[
            pl.BlockSpec((NC, ROWS, D), lambda i: (0, i, 0)),
            pl.BlockSpec((NC, ROWS, D), lambda i: (0, i, 0)),
            pl.BlockSpec((NC, ROWS, 16), lambda i: (0, i, 0)),
            pl.BlockSpec((NC, ROWS, 16), lambda i: (0, i, 0)),
            pl.BlockSpec((16, D), lambda i: (0, 0)),
            pl.BlockSpec((1, D), lambda i: (0, 0)),
            pl.BlockSpec((1, D), lambda i: (0, 0)),
            pl.BlockSpec((D, HID), lambda i: (0, 0)),
            pl.BlockSpec((1, HID), lambda i: (0, 0)),
            pl.BlockSpec((HID, 1), lambda i: (0, 0)),
        ],
        out_specs=[
            pl.BlockSpec((ROWS, D), lambda i: (i, 0)),
            pl.BlockSpec((ROWS, D), lambda i: (i, 0)),
            pl.BlockSpec((1, 1, 2), lambda i: (i, 0, 0)),
        ],
        out_shape=[
            jax.ShapeDtypeStruct((N, D), jnp.float32),
            jax.ShapeDtypeStruct((N, D), jnp.float32),
            jax.ShapeDtypeStruct((NBLK, 1, 2), jnp.float32),
        ],
    )(nums[0], nums[1], den0, den1, mexp, b0.reshape(1, D), b1.reshape(1, D),
      sW1, sb1.reshape(1, HID), sW2)

    wsum = jnp.sum(wparts[:, 0, :], axis=0, keepdims=True) / N

    out = pl.pallas_call(
        _combine_kernel,
        grid=(NBLK,),
        in_specs=[
            pl.BlockSpec((ROWS, D), lambda i: (i, 0)),
            pl.BlockSpec((ROWS, D), lambda i: (i, 0)),
            pl.BlockSpec((1, 2), lambda i: (0, 0)),
        ],
        out_specs=pl.BlockSpec((ROWS, D), lambda i: (i, 0)),
        out_shape=jax.ShapeDtypeStruct((N, D), jnp.float32),
    )(z0, z1, wsum)

    return out


# speedup vs baseline: 1.0511x; 1.0511x over previous
import functools

import numpy as np

import jax
import jax.numpy as jnp
from jax import lax
from jax.experimental import pallas as pl
from jax.experimental.pallas import tpu as pltpu
from jax.experimental.pallas import tpu_sc as plsc

N = 10000
E = 160000
IN = 256
HEADS = 8
OUT = 64
D = HEADS * OUT
HID = 64

ROWS = 400
NBLK = N // ROWS

NC = 2
NS = 16
NW = NC * NS
K = 128
EW = 5120
EPAD = NW * EW
NPAD = 10240
RPS = NPAD // NS
NCHUNK = EW // K

_MESH = plsc.VectorSubcoreMesh(core_axis_name="c", subcore_axis_name="s")
_SC_PARAMS = pltpu.CompilerParams(use_tc_tiling_on_sc=False,
                                  needs_layout_passes=False)


def _feat_elr_kernel(h_ref, w_ref, b_ref, feat_ref, elr_ref):
    f = jnp.dot(h_ref[...], w_ref[...], preferred_element_type=jnp.float32)
    feat_ref[...] = f
    elr_ref[...] = jnp.dot(f, b_ref[...], preferred_element_type=jnp.float32)


def _z_sem_kernel(n0_ref, n1_ref, d0_ref, d1_ref, m_ref, b0_ref, b1_ref,
                  sw1_ref, sb1_ref, sw2_ref, z0_ref, z1_ref, w_ref):
    num0 = n0_ref[0] + n0_ref[1]
    num1 = n1_ref[0] + n1_ref[1]
    den0 = d0_ref[0] + d0_ref[1] + 1e-9
    den1 = d1_ref[0] + d1_ref[1] + 1e-9
    dex0 = jnp.dot(1.0 / den0, m_ref[...], preferred_element_type=jnp.float32)
    dex1 = jnp.dot(1.0 / den1, m_ref[...], preferred_element_type=jnp.float32)
    z0 = num0 * dex0 + b0_ref[...]
    z1 = num1 * dex1 + b1_ref[...]
    z0 = jnp.where(z0 > 0, z0, jnp.exp(jnp.minimum(z0, 0.0)) - 1.0)
    z1 = jnp.where(z1 > 0, z1, jnp.exp(jnp.minimum(z1, 0.0)) - 1.0)
    z0_ref[...] = z0
    z1_ref[...] = z1
    t0 = jnp.dot(jnp.tanh(jnp.dot(z0, sw1_ref[...],
                                  preferred_element_type=jnp.float32)
                          + sb1_ref[...]),
                 sw2_ref[...], preferred_element_type=jnp.float32)
    t1 = jnp.dot(jnp.tanh(jnp.dot(z1, sw1_ref[...],
                                  preferred_element_type=jnp.float32)
                          + sb1_ref[...]),
                 sw2_ref[...], preferred_element_type=jnp.float32)
    w_ref[...] = jnp.concatenate(
        [jnp.sum(t0).reshape(1, 1, 1), jnp.sum(t1).reshape(1, 1, 1)], axis=2)


def _combine_kernel(z0_ref, z1_ref, w_ref, out_ref):
    w0 = w_ref[0, 0]
    w1 = w_ref[0, 1]
    m = jnp.maximum(w0, w1)
    e0 = jnp.exp(w0 - m)
    e1 = jnp.exp(w1 - m)
    beta0 = e0 / (e0 + e1)
    beta1 = e1 / (e0 + e1)
    out_ref[...] = beta0 * z0_ref[...] + beta1 * z1_ref[...]


def _edge_logits_body(ela_hbm, erb_hbm, src_hbm, dst_hbm, z16_hbm,
                      ee_hbm, den_hbm,
                      sidx, didx, ga, gb, eev, acc):
    cid = lax.axis_index("c")
    sid = lax.axis_index("s")
    wid = sid * NC + cid

    pltpu.sync_copy(z16_hbm, acc.at[pl.ds(sid * RPS, RPS)])
    plsc.subcore_barrier()

    @pl.loop(0, NCHUNK)
    def _(i):
        base = wid * EW + i * K
        pltpu.sync_copy(src_hbm.at[pl.ds(base, K)], sidx.at[0])
        pltpu.sync_copy(dst_hbm.at[pl.ds(base, K)], didx.at[0])
        pltpu.sync_copy(ela_hbm.at[sidx.at[0]], ga)
        pltpu.sync_copy(erb_hbm.at[didx.at[0]], gb)

        @pl.loop(0, K)
        def _(r):
            x = ga[r] + gb[r]
            x = jnp.maximum(x, 0.2 * x)
            eev[r] = jnp.exp(x)

        pltpu.sync_copy(eev, ee_hbm.at[pl.ds(base, K)])
        pltpu.sync_copy(eev, acc.at[didx.at[0]], add=True)

    plsc.subcore_barrier()
    pltpu.sync_copy(acc.at[pl.ds(sid * RPS, RPS)],
                    den_hbm.at[cid, pl.ds(sid * RPS, RPS)])


def _aggregate_body(f0_hbm, f1_hbm, f2_hbm, f3_hbm, src3_hbm, dst3_hbm,
                    ee_hbm, z128_hbm, num_hbm,
                    sidx, didx, g0, g1, ee0, ee1, sem0, sem1, acc):
    cid = lax.axis_index("c")
    sid = lax.axis_index("s")
    wid = sid * NC + cid

    pltpu.sync_copy(src3_hbm.at[wid], sidx)
    pltpu.sync_copy(dst3_hbm.at[wid], didx)

    feats = (f0_hbm, f1_hbm, f2_hbm, f3_hbm)
    bufs = ((g0, ee0, sem0), (g1, ee1, sem1))
    NBUF = 2

    for p in range(4):
        fp = feats[p]

        pltpu.sync_copy(z128_hbm, acc.at[pl.ds(sid * RPS, RPS)])
        plsc.subcore_barrier()

        def issue(i, b):
            gb, eb, sb = bufs[b]
            pltpu.async_copy(fp.at[sidx.at[i]], gb, sb)
            pltpu.async_copy(ee_hbm.at[pl.ds(wid * EW + i * K, K)], eb, sb)

        for b0 in range(NBUF):
            issue(b0, b0)

        @pl.loop(0, NCHUNK, step=NBUF)
        def _(i2):
            for b in range(NBUF):
                i = i2 + b
                gb, eb, sb = bufs[b]
                pltpu.make_async_copy(fp.at[sidx.at[i]], gb, sb).wait()
                pltpu.make_async_copy(
                    ee_hbm.at[pl.ds(wid * EW + i * K, K)], eb, sb).wait()

                @pl.loop(0, K)
                def _(j):
                    jv = jnp.full((16,), j, jnp.int32)
                    m0 = plsc.load_gather(
                        eb, [jv, jnp.full((16,), 2 * p, jnp.int32)])
                    m1 = plsc.load_gather(
                        eb, [jv, jnp.full((16,), 2 * p + 1, jnp.int32)])
                    for v in range(8):
                        m = m0 if v < 4 else m1
                        c = v * 16
                        gb[j, pl.ds(c, 16)] = gb[j, pl.ds(c, 16)] * m

                pltpu.sync_copy(gb, acc.at[didx.at[i]], add=True)

                @pl.when(i + NBUF < NCHUNK)
                def _():
                    issue(i + NBUF, b)

        plsc.subcore_barrier()
        pltpu.sync_copy(acc.at[pl.ds(sid * RPS, RPS)],
                        num_hbm.at[p, cid, pl.ds(sid * RPS, RPS)])
        plsc.subcore_barrier()


def _edge_logits(ela, erb, src, dst, z16):
    kern = pl.kernel(
        _edge_logits_body,
        out_type=(jax.ShapeDtypeStruct((EPAD, 16), jnp.float32),
                  jax.ShapeDtypeStruct((NC, NPAD, 16), jnp.float32)),
        mesh=_MESH,
        scratch_types=[
            pltpu.VMEM((1, K), jnp.int32),
            pltpu.VMEM((1, K), jnp.int32),
            pltpu.VMEM((K, 16), jnp.float32),
            pltpu.VMEM((K, 16), jnp.float32),
            pltpu.VMEM((K, 16), jnp.float32),
            pltpu.VMEM_SHARED((NPAD, 16), jnp.float32),
        ],
        compiler_params=_SC_PARAMS,
    )
    return kern(ela, erb, src, dst, z16)


def _aggregate(f0, f1, f2, f3, src3, dst3, ee, z128):
    kern = pl.kernel(
        _aggregate_body,
        out_type=jax.ShapeDtypeStruct((4, NC, NPAD, 128), jnp.float32),
        mesh=_MESH,
        scratch_types=[
            pltpu.VMEM((NCHUNK, K), jnp.int32),
            pltpu.VMEM((NCHUNK, K), jnp.int32),
            pltpu.VMEM((K, 128), jnp.float32),
            pltpu.VMEM((K, 128), jnp.float32),
            pltpu.VMEM((K, 16), jnp.float32),
            pltpu.VMEM((K, 16), jnp.float32),
            pltpu.SemaphoreType.DMA,
            pltpu.SemaphoreType.DMA,
            pltpu.VMEM_SHARED((NPAD, 128), jnp.float32),
        ],
        compiler_params=_SC_PARAMS,
    )
    return kern(f0, f1, f2, f3, src3, dst3, ee, z128)


def _block_diag(a):
    return (jnp.eye(HEADS, dtype=a.dtype)[:, None, :] * a[:, :, None]).reshape(D, HEADS)


def kernel(H, edge_index0, edge_index1, W0, al0, ar0, b0, W1, al1, ar1, b1, sW1, sb1, sW2):
    h = H[0]
    wcat = jnp.concatenate([W0, W1], axis=1)

    bmat = jnp.zeros((2 * D, 64), dtype=jnp.float32)
    bmat = bmat.at[:D, 0:HEADS].set(_block_diag(al0))
    bmat = bmat.at[:D, 16:16 + HEADS].set(_block_diag(ar0))
    bmat = bmat.at[D:, 32:32 + HEADS].set(_block_diag(al1))
    bmat = bmat.at[D:, 48:48 + HEADS].set(_block_diag(ar1))

    feat01, elr = pl.pallas_call(
        _feat_elr_kernel,
        grid=(NBLK,),
        in_specs=[
            pl.BlockSpec((ROWS, IN), lambda i: (i, 0)),
            pl.BlockSpec((IN, 2 * D), lambda i: (0, 0)),
            pl.BlockSpec((2 * D, 64), lambda i: (0, 0)),
        ],
        out_specs=[
            pl.BlockSpec((ROWS, 2 * D), lambda i: (i, 0)),
            pl.BlockSpec((ROWS, 64), lambda i: (i, 0)),
        ],
        out_shape=[
            jax.ShapeDtypeStruct((N, 2 * D), jnp.float32),
            jax.ShapeDtypeStruct((N, 64), jnp.float32),
        ],
    )(h, wcat, bmat)

    ela0 = elr[:, 0:16]
    erb0 = jnp.pad(elr[:, 16:32], ((0, NPAD - N), (0, 0)))
    ela1 = elr[:, 32:48]
    erb1 = jnp.pad(elr[:, 48:64], ((0, NPAD - N), (0, 0)))

    pad = EPAD - E
    i32 = jnp.int32
    sink = N + jnp.arange(pad, dtype=i32) % (NPAD - N)
    src0 = jnp.concatenate([edge_index0[0], jnp.zeros((pad,), i32)])
    dst0 = jnp.concatenate([edge_index0[1], sink])
    src1 = jnp.concatenate([edge_index1[0], jnp.zeros((pad,), i32)])
    dst1 = jnp.concatenate([edge_index1[1], sink])

    z16 = jnp.zeros((RPS, 16), jnp.float32)
    z128 = jnp.zeros((RPS, 128), jnp.float32)

    ee0, den0 = _edge_logits(ela0, erb0, src0, dst0, z16)
    ee1, den1 = _edge_logits(ela1, erb1, src1, dst1, z16)

    nums = []
    for m, (srcm, dstm, eem) in enumerate(((src0, dst0, ee0),
                                           (src1, dst1, ee1))):
        fs = [lax.slice(feat01, (0, m * D + p * 128),
                        (N, m * D + (p + 1) * 128)) for p in range(4)]
        num4 = _aggregate(fs[0], fs[1], fs[2], fs[3],
                          srcm.reshape(NW, NCHUNK, K),
                          dstm.reshape(NW, NCHUNK, K), eem, z128)
        nums.append(jnp.concatenate(
            [num4[p] for p in range(4)], axis=2)[:, :N, :])

    den0 = den0[:, :N, :]
    den1 = den1[:, :N, :]

    mexp = jnp.zeros((16, D), jnp.float32)
    for hh in range(HEADS):
        mexp = mexp.at[hh, hh * OUT:(hh + 1) * OUT].set(1.0)

    z0, z1, wparts = pl.pallas_call(
        _z_sem_kernel,
        grid=(NBLK,),
        in_specs=[
            pl.BlockSpec((NC, ROWS, D), lambda i: (0, i, 0)),
            pl.BlockSpec((NC, ROWS, D), lambda i: (0, i, 0)),
            pl.BlockSpec((NC, ROWS, 16), lambda i: (0, i, 0)),
            pl.BlockSpec((NC, ROWS, 16), lambda i: (0, i, 0)),
            pl.BlockSpec((16, D), lambda i: (0, 0)),
            pl.BlockSpec((1, D), lambda i: (0, 0)),
            pl.BlockSpec((1, D), lambda i: (0, 0)),
            pl.BlockSpec((D, HID), lambda i: (0, 0)),
            pl.BlockSpec((1, HID), lambda i: (0, 0)),
            pl.BlockSpec((HID, 1), lambda i: (0, 0)),
        ],
        out_specs=[
            pl.BlockSpec((ROWS, D), lambda i: (i, 0)),
            pl.BlockSpec((ROWS, D), lambda i: (i, 0)),
            pl.BlockSpec((1, 1, 2), lambda i: (i, 0, 0)),
        ],
        out_shape=[
            jax.ShapeDtypeStruct((N, D), jnp.float32),
            jax.ShapeDtypeStruct((N, D), jnp.float32),
            jax.ShapeDtypeStruct((NBLK, 1, 2), jnp.float32),
        ],
    )(nums[0], nums[1], den0, den1, mexp, b0.reshape(1, D),
      b1.reshape(1, D), sW1, sb1.reshape(1, HID), sW2)

    wsum = jnp.sum(wparts[:, 0, :], axis=0, keepdims=True) / N

    out = pl.pallas_call(
        _combine_kernel,
        grid=(NBLK,),
        in_specs=[
            pl.BlockSpec((ROWS, D), lambda i: (i, 0)),
            pl.BlockSpec((ROWS, D), lambda i: (i, 0)),
            pl.BlockSpec((1, 2), lambda i: (0, 0)),
        ],
        out_specs=pl.BlockSpec((ROWS, D), lambda i: (i, 0)),
        out_shape=jax.ShapeDtypeStruct((N, D), jnp.float32),
    )(z0, z1, wsum)

    return out


# speedup vs baseline: 1.1299x; 1.0750x over previous
import functools

import numpy as np

import jax
import jax.numpy as jnp
from jax import lax
from jax.experimental import pallas as pl
from jax.experimental.pallas import tpu as pltpu
from jax.experimental.pallas import tpu_sc as plsc

N = 10000
E = 160000
IN = 256
HEADS = 8
OUT = 64
D = HEADS * OUT
HID = 64

ROWS = 400
NBLK = N // ROWS

NC = 2
NS = 16
NW = NC * NS
K = 128
EW = 5120
EPAD = NW * EW
NPAD = 10240
RPS = NPAD // NS
NCHUNK = EW // K

_MESH = plsc.VectorSubcoreMesh(core_axis_name="c", subcore_axis_name="s")
_SC_PARAMS = pltpu.CompilerParams(use_tc_tiling_on_sc=False,
                                  needs_layout_passes=False)


def _feat_elr_kernel(h_ref, w_ref, b_ref, feat_ref, elr_ref):
    f = jnp.dot(h_ref[...], w_ref[...], preferred_element_type=jnp.float32)
    feat_ref[...] = f
    elr_ref[...] = jnp.dot(f, b_ref[...], preferred_element_type=jnp.float32)


def _z_sem_kernel(n0_ref, n1_ref, d0_ref, d1_ref, m_ref, b0_ref, b1_ref,
                  sw1_ref, sb1_ref, sw2_ref, z0_ref, z1_ref, w_ref):
    num0 = n0_ref[0] + n0_ref[1]
    num1 = n1_ref[0] + n1_ref[1]
    den0 = d0_ref[0] + d0_ref[1] + 1e-9
    den1 = d1_ref[0] + d1_ref[1] + 1e-9
    dex0 = jnp.dot(1.0 / den0, m_ref[...], preferred_element_type=jnp.float32)
    dex1 = jnp.dot(1.0 / den1, m_ref[...], preferred_element_type=jnp.float32)
    z0 = num0 * dex0 + b0_ref[...]
    z1 = num1 * dex1 + b1_ref[...]
    z0 = jnp.where(z0 > 0, z0, jnp.exp(jnp.minimum(z0, 0.0)) - 1.0)
    z1 = jnp.where(z1 > 0, z1, jnp.exp(jnp.minimum(z1, 0.0)) - 1.0)
    z0_ref[...] = z0
    z1_ref[...] = z1
    t0 = jnp.dot(jnp.tanh(jnp.dot(z0, sw1_ref[...],
                                  preferred_element_type=jnp.float32)
                          + sb1_ref[...]),
                 sw2_ref[...], preferred_element_type=jnp.float32)
    t1 = jnp.dot(jnp.tanh(jnp.dot(z1, sw1_ref[...],
                                  preferred_element_type=jnp.float32)
                          + sb1_ref[...]),
                 sw2_ref[...], preferred_element_type=jnp.float32)
    w_ref[...] = jnp.concatenate(
        [jnp.sum(t0).reshape(1, 1, 1), jnp.sum(t1).reshape(1, 1, 1)], axis=2)


def _combine_kernel(z0_ref, z1_ref, w_ref, out_ref):
    w0 = w_ref[0, 0]
    w1 = w_ref[0, 1]
    m = jnp.maximum(w0, w1)
    e0 = jnp.exp(w0 - m)
    e1 = jnp.exp(w1 - m)
    beta0 = e0 / (e0 + e1)
    beta1 = e1 / (e0 + e1)
    out_ref[...] = beta0 * z0_ref[...] + beta1 * z1_ref[...]


def _edge_logits_body(ela_hbm, erb_hbm, src3_hbm, dst3_hbm, z16_hbm,
                      ee_hbm, den_hbm,
                      sidx, didx, ga0, gb0, ga1, gb1, ee0, ee1,
                      sem0, sem1, acc):
    cid = lax.axis_index("c")
    sid = lax.axis_index("s")
    wid = sid * NC + cid

    pltpu.sync_copy(src3_hbm.at[wid], sidx)
    pltpu.sync_copy(dst3_hbm.at[wid], didx)
    pltpu.sync_copy(z16_hbm, acc.at[pl.ds(sid * RPS, RPS)])
    plsc.subcore_barrier()

    bufs = ((ga0, gb0, ee0, sem0), (ga1, gb1, ee1, sem1))

    def issue(i, b):
        gab, gbb, _, sb = bufs[b]
        pltpu.async_copy(ela_hbm.at[sidx.at[i]], gab, sb)
        pltpu.async_copy(erb_hbm.at[didx.at[i]], gbb, sb)

    issue(0, 0)
    issue(1, 1)

    @pl.loop(0, NCHUNK, step=2)
    def _(i2):
        for b in range(2):
            i = i2 + b
            gab, gbb, eev, sb = bufs[b]
            pltpu.make_async_copy(ela_hbm.at[sidx.at[i]], gab, sb).wait()
            pltpu.make_async_copy(erb_hbm.at[didx.at[i]], gbb, sb).wait()

            @pl.loop(0, K)
            def _(r):
                x = gab[r] + gbb[r]
                x = jnp.maximum(x, 0.2 * x)
                eev[r] = jnp.exp(x)

            pltpu.sync_copy(eev, ee_hbm.at[pl.ds(wid * EW + i * K, K)])
            pltpu.sync_copy(eev, acc.at[didx.at[i]], add=True)

            @pl.when(i + 2 < NCHUNK)
            def _():
                issue(i + 2, b)

    plsc.subcore_barrier()
    pltpu.sync_copy(acc.at[pl.ds(sid * RPS, RPS)],
                    den_hbm.at[cid, pl.ds(sid * RPS, RPS)])


def _aggregate_body(f0_hbm, f1_hbm, f2_hbm, f3_hbm, src3_hbm, dst3_hbm,
                    ee_hbm, z128_hbm, num_hbm,
                    sidx, didx, g0, g1, ee0, ee1, sem0, sem1, acc):
    cid = lax.axis_index("c")
    sid = lax.axis_index("s")
    wid = sid * NC + cid

    pltpu.sync_copy(src3_hbm.at[wid], sidx)
    pltpu.sync_copy(dst3_hbm.at[wid], didx)

    feats = (f0_hbm, f1_hbm, f2_hbm, f3_hbm)
    bufs = ((g0, ee0, sem0), (g1, ee1, sem1))
    NBUF = 2

    for p in range(4):
        fp = feats[p]

        pltpu.sync_copy(z128_hbm, acc.at[pl.ds(sid * RPS, RPS)])
        plsc.subcore_barrier()

        def issue(i, b):
            gb, eb, sb = bufs[b]
            pltpu.async_copy(fp.at[sidx.at[i]], gb, sb)
            pltpu.async_copy(ee_hbm.at[pl.ds(wid * EW + i * K, K)], eb, sb)

        for b0 in range(NBUF):
            issue(b0, b0)

        @pl.loop(0, NCHUNK, step=NBUF)
        def _(i2):
            for b in range(NBUF):
                i = i2 + b
                gb, eb, sb = bufs[b]
                pltpu.make_async_copy(fp.at[sidx.at[i]], gb, sb).wait()
                pltpu.make_async_copy(
                    ee_hbm.at[pl.ds(wid * EW + i * K, K)], eb, sb).wait()

                @pl.loop(0, K)
                def _(j):
                    jv = jnp.full((16,), j, jnp.int32)
                    m0 = plsc.load_gather(
                        eb, [jv, jnp.full((16,), 2 * p, jnp.int32)])
                    m1 = plsc.load_gather(
                        eb, [jv, jnp.full((16,), 2 * p + 1, jnp.int32)])
                    for v in range(8):
                        m = m0 if v < 4 else m1
                        c = v * 16
                        gb[j, pl.ds(c, 16)] = gb[j, pl.ds(c, 16)] * m

                pltpu.sync_copy(gb, acc.at[didx.at[i]], add=True)

                @pl.when(i + NBUF < NCHUNK)
                def _():
                    issue(i + NBUF, b)

        plsc.subcore_barrier()
        pltpu.sync_copy(acc.at[pl.ds(sid * RPS, RPS)],
                        num_hbm.at[p, cid, pl.ds(sid * RPS, RPS)])
        plsc.subcore_barrier()


def _edge_logits(ela, erb, src3, dst3, z16):
    kern = pl.kernel(
        _edge_logits_body,
        out_type=(jax.ShapeDtypeStruct((EPAD, 16), jnp.float32),
                  jax.ShapeDtypeStruct((NC, NPAD, 16), jnp.float32)),
        mesh=_MESH,
        scratch_types=[
            pltpu.VMEM((NCHUNK, K), jnp.int32),
            pltpu.VMEM((NCHUNK, K), jnp.int32),
            pltpu.VMEM((K, 16), jnp.float32),
            pltpu.VMEM((K, 16), jnp.float32),
            pltpu.VMEM((K, 16), jnp.float32),
            pltpu.VMEM((K, 16), jnp.float32),
            pltpu.VMEM((K, 16), jnp.float32),
            pltpu.VMEM((K, 16), jnp.float32),
            pltpu.SemaphoreType.DMA,
            pltpu.SemaphoreType.DMA,
            pltpu.VMEM_SHARED((NPAD, 16), jnp.float32),
        ],
        compiler_params=_SC_PARAMS,
    )
    return kern(ela, erb, src3, dst3, z16)


def _aggregate(f0, f1, f2, f3, src3, dst3, ee, z128):
    kern = pl.kernel(
        _aggregate_body,
        out_type=jax.ShapeDtypeStruct((4, NC, NPAD, 128), jnp.float32),
        mesh=_MESH,
        scratch_types=[
            pltpu.VMEM((NCHUNK, K), jnp.int32),
            pltpu.VMEM((NCHUNK, K), jnp.int32),
            pltpu.VMEM((K, 128), jnp.float32),
            pltpu.VMEM((K, 128), jnp.float32),
            pltpu.VMEM((K, 16), jnp.float32),
            pltpu.VMEM((K, 16), jnp.float32),
            pltpu.SemaphoreType.DMA,
            pltpu.SemaphoreType.DMA,
            pltpu.VMEM_SHARED((NPAD, 128), jnp.float32),
        ],
        compiler_params=_SC_PARAMS,
    )
    return kern(f0, f1, f2, f3, src3, dst3, ee, z128)


def _block_diag(a):
    return (jnp.eye(HEADS, dtype=a.dtype)[:, None, :] * a[:, :, None]).reshape(D, HEADS)


def kernel(H, edge_index0, edge_index1, W0, al0, ar0, b0, W1, al1, ar1, b1, sW1, sb1, sW2):
    h = H[0]
    wcat = jnp.concatenate([W0, W1], axis=1)

    bmat = jnp.zeros((2 * D, 64), dtype=jnp.float32)
    bmat = bmat.at[:D, 0:HEADS].set(_block_diag(al0))
    bmat = bmat.at[:D, 16:16 + HEADS].set(_block_diag(ar0))
    bmat = bmat.at[D:, 32:32 + HEADS].set(_block_diag(al1))
    bmat = bmat.at[D:, 48:48 + HEADS].set(_block_diag(ar1))

    feat01, elr = pl.pallas_call(
        _feat_elr_kernel,
        grid=(NBLK,),
        in_specs=[
            pl.BlockSpec((ROWS, IN), lambda i: (i, 0)),
            pl.BlockSpec((IN, 2 * D), lambda i: (0, 0)),
            pl.BlockSpec((2 * D, 64), lambda i: (0, 0)),
        ],
        out_specs=[
            pl.BlockSpec((ROWS, 2 * D), lambda i: (i, 0)),
            pl.BlockSpec((ROWS, 64), lambda i: (i, 0)),
        ],
        out_shape=[
            jax.ShapeDtypeStruct((N, 2 * D), jnp.float32),
            jax.ShapeDtypeStruct((N, 64), jnp.float32),
        ],
    )(h, wcat, bmat)

    ela0 = elr[:, 0:16]
    erb0 = jnp.pad(elr[:, 16:32], ((0, NPAD - N), (0, 0)))
    ela1 = elr[:, 32:48]
    erb1 = jnp.pad(elr[:, 48:64], ((0, NPAD - N), (0, 0)))

    pad = EPAD - E
    i32 = jnp.int32
    sink = N + jnp.arange(pad, dtype=i32) % (NPAD - N)
    src0 = jnp.concatenate([edge_index0[0], jnp.zeros((pad,), i32)])
    dst0 = jnp.concatenate([edge_index0[1], sink])
    src1 = jnp.concatenate([edge_index1[0], jnp.zeros((pad,), i32)])
    dst1 = jnp.concatenate([edge_index1[1], sink])

    z16 = jnp.zeros((RPS, 16), jnp.float32)
    z128 = jnp.zeros((RPS, 128), jnp.float32)

    src0 = src0.reshape(NW, NCHUNK, K)
    dst0 = dst0.reshape(NW, NCHUNK, K)
    src1 = src1.reshape(NW, NCHUNK, K)
    dst1 = dst1.reshape(NW, NCHUNK, K)

    ee0, den0 = _edge_logits(ela0, erb0, src0, dst0, z16)
    ee1, den1 = _edge_logits(ela1, erb1, src1, dst1, z16)

    nums = []
    for m, (srcm, dstm, eem) in enumerate(((src0, dst0, ee0),
                                           (src1, dst1, ee1))):
        fs = [lax.slice(feat01, (0, m * D + p * 128),
                        (N, m * D + (p + 1) * 128)) for p in range(4)]
        num4 = _aggregate(fs[0], fs[1], fs[2], fs[3], srcm, dstm, eem, z128)
        nums.append(jnp.concatenate(
            [num4[p] for p in range(4)], axis=2)[:, :N, :])

    den0 = den0[:, :N, :]
    den1 = den1[:, :N, :]

    mexp = jnp.zeros((16, D), jnp.float32)
    for hh in range(HEADS):
        mexp = mexp.at[hh, hh * OUT:(hh + 1) * OUT].set(1.0)

    z0, z1, wparts = pl.pallas_call(
        _z_sem_kernel,
        grid=(NBLK,),
        in_specs=[
            pl.BlockSpec((NC, ROWS, D), lambda i: (0, i, 0)),
            pl.BlockSpec((NC, ROWS, D), lambda i: (0, i, 0)),
            pl.BlockSpec((NC, ROWS, 16), lambda i: (0, i, 0)),
            pl.BlockSpec((NC, ROWS, 16), lambda i: (0, i, 0)),
            pl.BlockSpec((16, D), lambda i: (0, 0)),
            pl.BlockSpec((1, D), lambda i: (0, 0)),
            pl.BlockSpec((1, D), lambda i: (0, 0)),
            pl.BlockSpec((D, HID), lambda i: (0, 0)),
            pl.BlockSpec((1, HID), lambda i: (0, 0)),
            pl.BlockSpec((HID, 1), lambda i: (0, 0)),
        ],
        out_specs=[
            pl.BlockSpec((ROWS, D), lambda i: (i, 0)),
            pl.BlockSpec((ROWS, D), lambda i: (i, 0)),
            pl.BlockSpec((1, 1, 2), lambda i: (i, 0, 0)),
        ],
        out_shape=[
            jax.ShapeDtypeStruct((N, D), jnp.float32),
            jax.ShapeDtypeStruct((N, D), jnp.float32),
            jax.ShapeDtypeStruct((NBLK, 1, 2), jnp.float32),
        ],
    )(nums[0], nums[1], den0, den1, mexp, b0.reshape(1, D),
      b1.reshape(1, D), sW1, sb1.reshape(1, HID), sW2)

    wsum = jnp.sum(wparts[:, 0, :], axis=0, keepdims=True) / N

    out = pl.pallas_call(
        _combine_kernel,
        grid=(NBLK,),
        in_specs=[
            pl.BlockSpec((ROWS, D), lambda i: (i, 0)),
            pl.BlockSpec((ROWS, D), lambda i: (i, 0)),
            pl.BlockSpec((1, 2), lambda i: (0, 0)),
        ],
        out_specs=pl.BlockSpec((ROWS, D), lambda i: (i, 0)),
        out_shape=jax.ShapeDtypeStruct((N, D), jnp.float32),
    )(z0, z1, wsum)

    return out
